# baseline (speedup 1.0000x reference)
import functools

import jax
import jax.numpy as jnp
from jax import lax
from jax.experimental import pallas as pl
from jax.experimental.pallas import tpu as pltpu
from jax.experimental.pallas import tpu_sc as plsc

B, N, D = 32, 1024, 3
L = 16
NCHUNK = N // L
LB = 4

B_SC = 2
W_SUB = 16
CPW = NCHUNK // W_SUB

_INF = float("inf")


def _direction_pass(scal_v, vec_v, colmin_v, base):
  def outer(js, _):
    soff = js * L
    c0 = scal_v[0, pl.ds(soff, L)]
    c1 = scal_v[1, pl.ds(soff, L)]
    c2 = scal_v[2, pl.ds(soff, L)]
    a0_all = c0 * -2.0
    a1_all = c1 * -2.0
    a2_all = c2 * -2.0
    ss_all = c0 * c0 + c1 * c1 + c2 * c2

    for lb in range(L // LB):
      a0s = [jnp.full((L,), a0_all[lb * LB + i], jnp.float32)
             for i in range(LB)]
      a1s = [jnp.full((L,), a1_all[lb * LB + i], jnp.float32)
             for i in range(LB)]
      a2s = [jnp.full((L,), a2_all[lb * LB + i], jnp.float32)
             for i in range(LB)]
      sss = [jnp.full((L,), ss_all[lb * LB + i], jnp.float32)
             for i in range(LB)]

      for j in range(CPW):
        off = base + j * L
        v0 = vec_v[0, pl.ds(off, L)]
        v1 = vec_v[1, pl.ds(off, L)]
        v2 = vec_v[2, pl.ds(off, L)]
        cm = colmin_v[pl.ds(off, L)]
        for i in range(LB):
          t = sss[i] + a0s[i] * v0 + a1s[i] * v1 + a2s[i] * v2
          cm = jnp.minimum(cm, t)
        colmin_v[pl.ds(off, L)] = cm
    return 0

  lax.fori_loop(0, NCHUNK, outer, 0)


def _sc_body(tmpl_hbm, src_hbm, out_hbm, tmpl_v, src_v, colmin0_v, colmin1_v,
             out_v):
  core = lax.axis_index("c")
  sub = lax.axis_index("s")
  base = sub * (CPW * L)

  pltpu.sync_copy(tmpl_hbm.at[core], tmpl_v)
  pltpu.sync_copy(src_hbm.at[core], src_v)

  for j in range(CPW):
    off = base + j * L
    colmin0_v[pl.ds(off, L)] = jnp.full((L,), _INF, jnp.float32)
    colmin1_v[pl.ds(off, L)] = jnp.full((L,), _INF, jnp.float32)

  _direction_pass(src_v, tmpl_v, colmin0_v, base)
  _direction_pass(tmpl_v, src_v, colmin1_v, base)

  vacc = jnp.zeros((L,), jnp.float32)
  for j in range(CPW):
    off = base + j * L
    t0 = tmpl_v[0, pl.ds(off, L)]
    t1 = tmpl_v[1, pl.ds(off, L)]
    t2 = tmpl_v[2, pl.ds(off, L)]
    s0 = src_v[0, pl.ds(off, L)]
    s1 = src_v[1, pl.ds(off, L)]
    s2 = src_v[2, pl.ds(off, L)]
    min0 = colmin0_v[pl.ds(off, L)] + (t0 * t0 + t1 * t1 + t2 * t2)
    min1 = colmin1_v[pl.ds(off, L)] + (s0 * s0 + s1 * s1 + s2 * s2)
    vacc = vacc + (min0 + min1)

  out_v[...] = vacc
  pltpu.sync_copy(out_v, out_hbm.at[core, sub])


def _sc_chamfer(tmpl_t, src_t):
  mesh = plsc.VectorSubcoreMesh(core_axis_name="c", subcore_axis_name="s")
  run = pl.kernel(
      _sc_body,
      out_type=jax.ShapeDtypeStruct((B_SC, W_SUB, L), jnp.float32),
      mesh=mesh,
      scratch_types=[
          pltpu.VMEM((D, N), jnp.float32),
          pltpu.VMEM((D, N), jnp.float32),
          pltpu.VMEM((N,), jnp.float32),
          pltpu.VMEM((N,), jnp.float32),
          pltpu.VMEM((L,), jnp.float32),
      ],
  )
  out = run(tmpl_t, src_t)
  return jnp.sum(out, axis=(1, 2)) * (1.0 / N)


TB = 8


def _tc_body(x_ref, y_ref, out_ref):
  step = pl.program_id(0)
  acc = jnp.float32(0.0)
  for b in range(TB):
    lhs = x_ref[b]
    rhs = y_ref[b]

    r0 = lax.dot_general(lhs, rhs, (((1,), (1,)), ((), ())),
                         preferred_element_type=jnp.float32)

    m0 = jnp.min(r0, axis=1)
    m1 = jnp.min(r0, axis=0)
    acc = acc + (jnp.sum(m0) + jnp.sum(m1))

  scaled = acc * (1.0 / (N * B))

  @pl.when(step == 0)
  def _init():
    out_ref[0, 0, 0] = scaled

  @pl.when(step != 0)
  def _accum():
    out_ref[0, 0, 0] = out_ref[0, 0, 0] + scaled


def _tc_chamfer_mean(x, y):
  nb = x.shape[0]
  steps = nb // TB

  xx = jnp.sum(x * x, axis=2, keepdims=True)
  yy = jnp.sum(y * y, axis=2, keepdims=True)
  one = jnp.ones((nb, N, 1), jnp.float32)
  zero3 = jnp.zeros((nb, N, 3), jnp.float32)
  lhs = jnp.concatenate([x * -2.0, one, xx, zero3], axis=2)
  rhs = jnp.concatenate([y, yy, one, zero3], axis=2)

  out = pl.pallas_call(
      _tc_body,
      grid=(steps,),
      in_specs=[
          pl.BlockSpec((TB, N, 8), lambda i: (i, 0, 0)),
          pl.BlockSpec((TB, N, 8), lambda i: (i, 0, 0)),
      ],
      out_specs=pl.BlockSpec((1, 1, 1), lambda i: (0, 0, 0),
                             memory_space=pltpu.SMEM),
      out_shape=jax.ShapeDtypeStruct((1, 1, 1), jnp.float32),
      compiler_params=pltpu.CompilerParams(
          dimension_semantics=("arbitrary",),
      ),
  )(lhs, rhs)
  return jnp.reshape(out, ())


@jax.jit
def kernel(template, source):
  return _tc_chamfer_mean(template, source)


# speedup vs baseline: 4.2120x; 4.2120x over previous
import functools

import jax
import jax.numpy as jnp
from jax import lax
from jax.experimental import pallas as pl
from jax.experimental.pallas import tpu as pltpu
from jax.experimental.pallas import tpu_sc as plsc

B, N, D = 32, 1024, 3
L = 16
NCHUNK = N // L
LB = 4

B_SC = 2
W_SUB = 16
CPW = NCHUNK // W_SUB

_INF = float("inf")


def _direction_pass(scal_v, vec_v, colmin_v, base):
  def outer(js, _):
    soff = js * L
    c0 = scal_v[0, pl.ds(soff, L)]
    c1 = scal_v[1, pl.ds(soff, L)]
    c2 = scal_v[2, pl.ds(soff, L)]
    a0_all = c0 * -2.0
    a1_all = c1 * -2.0
    a2_all = c2 * -2.0
    ss_all = c0 * c0 + c1 * c1 + c2 * c2

    for lb in range(L // LB):
      a0s = [jnp.full((L,), a0_all[lb * LB + i], jnp.float32)
             for i in range(LB)]
      a1s = [jnp.full((L,), a1_all[lb * LB + i], jnp.float32)
             for i in range(LB)]
      a2s = [jnp.full((L,), a2_all[lb * LB + i], jnp.float32)
             for i in range(LB)]
      sss = [jnp.full((L,), ss_all[lb * LB + i], jnp.float32)
             for i in range(LB)]

      for j in range(CPW):
        off = base + j * L
        v0 = vec_v[0, pl.ds(off, L)]
        v1 = vec_v[1, pl.ds(off, L)]
        v2 = vec_v[2, pl.ds(off, L)]
        cm = colmin_v[pl.ds(off, L)]
        for i in range(LB):
          t = sss[i] + a0s[i] * v0 + a1s[i] * v1 + a2s[i] * v2
          cm = jnp.minimum(cm, t)
        colmin_v[pl.ds(off, L)] = cm
    return 0

  lax.fori_loop(0, NCHUNK, outer, 0)


def _sc_body(tmpl_hbm, src_hbm, out_hbm, tmpl_v, src_v, colmin0_v, colmin1_v,
             out_v):
  core = lax.axis_index("c")
  sub = lax.axis_index("s")
  base = sub * (CPW * L)

  pltpu.sync_copy(tmpl_hbm.at[core], tmpl_v)
  pltpu.sync_copy(src_hbm.at[core], src_v)

  for j in range(CPW):
    off = base + j * L
    colmin0_v[pl.ds(off, L)] = jnp.full((L,), _INF, jnp.float32)
    colmin1_v[pl.ds(off, L)] = jnp.full((L,), _INF, jnp.float32)

  _direction_pass(src_v, tmpl_v, colmin0_v, base)
  _direction_pass(tmpl_v, src_v, colmin1_v, base)

  vacc = jnp.zeros((L,), jnp.float32)
  for j in range(CPW):
    off = base + j * L
    t0 = tmpl_v[0, pl.ds(off, L)]
    t1 = tmpl_v[1, pl.ds(off, L)]
    t2 = tmpl_v[2, pl.ds(off, L)]
    s0 = src_v[0, pl.ds(off, L)]
    s1 = src_v[1, pl.ds(off, L)]
    s2 = src_v[2, pl.ds(off, L)]
    min0 = colmin0_v[pl.ds(off, L)] + (t0 * t0 + t1 * t1 + t2 * t2)
    min1 = colmin1_v[pl.ds(off, L)] + (s0 * s0 + s1 * s1 + s2 * s2)
    vacc = vacc + (min0 + min1)

  out_v[...] = vacc
  pltpu.sync_copy(out_v, out_hbm.at[core, sub])


def _sc_chamfer(tmpl_t, src_t):
  mesh = plsc.VectorSubcoreMesh(core_axis_name="c", subcore_axis_name="s")
  run = pl.kernel(
      _sc_body,
      out_type=jax.ShapeDtypeStruct((B_SC, W_SUB, L), jnp.float32),
      mesh=mesh,
      scratch_types=[
          pltpu.VMEM((D, N), jnp.float32),
          pltpu.VMEM((D, N), jnp.float32),
          pltpu.VMEM((N,), jnp.float32),
          pltpu.VMEM((N,), jnp.float32),
          pltpu.VMEM((L,), jnp.float32),
      ],
  )
  out = run(tmpl_t, src_t)
  return jnp.sum(out, axis=(1, 2)) * (1.0 / N)


TB = 8


def _tc_body(x_ref, y_ref, out_ref):
  step = pl.program_id(0)
  acc = jnp.float32(0.0)
  for b in range(TB):
    lhs = x_ref[b]
    rhs = y_ref[b]

    r0 = lax.dot_general(lhs, rhs, (((0,), (0,)), ((), ())),
                         preferred_element_type=jnp.float32)

    m0 = jnp.min(r0, axis=1)
    m1 = jnp.min(r0, axis=0)
    acc = acc + (jnp.sum(m0) + jnp.sum(m1))

  scaled = acc * (1.0 / (N * B))

  @pl.when(step == 0)
  def _init():
    out_ref[0, 0, 0] = scaled

  @pl.when(step != 0)
  def _accum():
    out_ref[0, 0, 0] = out_ref[0, 0, 0] + scaled


def _tc_chamfer_mean(x, y):
  nb = x.shape[0]
  steps = nb // TB

  xt = jnp.transpose(x, (0, 2, 1))
  yt = jnp.transpose(y, (0, 2, 1))
  xx = jnp.sum(xt * xt, axis=1, keepdims=True)
  yy = jnp.sum(yt * yt, axis=1, keepdims=True)
  one = jnp.ones((nb, 1, N), jnp.float32)
  zero3 = jnp.zeros((nb, 3, N), jnp.float32)
  lhs = jnp.concatenate([xt * -2.0, one, xx, zero3], axis=1)
  rhs = jnp.concatenate([yt, yy, one, zero3], axis=1)

  out = pl.pallas_call(
      _tc_body,
      grid=(steps,),
      in_specs=[
          pl.BlockSpec((TB, 8, N), lambda i: (i, 0, 0)),
          pl.BlockSpec((TB, 8, N), lambda i: (i, 0, 0)),
      ],
      out_specs=pl.BlockSpec((1, 1, 1), lambda i: (0, 0, 0),
                             memory_space=pltpu.SMEM),
      out_shape=jax.ShapeDtypeStruct((1, 1, 1), jnp.float32),
      compiler_params=pltpu.CompilerParams(
          dimension_semantics=("arbitrary",),
      ),
  )(lhs, rhs)
  return jnp.reshape(out, ())


@jax.jit
def kernel(template, source):
  return _tc_chamfer_mean(template, source)


# speedup vs baseline: 4.3315x; 1.0284x over previous
import functools

import jax
import jax.numpy as jnp
from jax import lax
from jax.experimental import pallas as pl
from jax.experimental.pallas import tpu as pltpu
from jax.experimental.pallas import tpu_sc as plsc

B, N, D = 32, 1024, 3
L = 16
NCHUNK = N // L
LB = 4

B_SC = 2
W_SUB = 16
CPW = NCHUNK // W_SUB

_INF = float("inf")


def _direction_pass(scal_v, vec_v, colmin_v, base):
  def outer(js, _):
    soff = js * L
    c0 = scal_v[0, pl.ds(soff, L)]
    c1 = scal_v[1, pl.ds(soff, L)]
    c2 = scal_v[2, pl.ds(soff, L)]
    a0_all = c0 * -2.0
    a1_all = c1 * -2.0
    a2_all = c2 * -2.0
    ss_all = c0 * c0 + c1 * c1 + c2 * c2

    for lb in range(L // LB):
      a0s = [jnp.full((L,), a0_all[lb * LB + i], jnp.float32)
             for i in range(LB)]
      a1s = [jnp.full((L,), a1_all[lb * LB + i], jnp.float32)
             for i in range(LB)]
      a2s = [jnp.full((L,), a2_all[lb * LB + i], jnp.float32)
             for i in range(LB)]
      sss = [jnp.full((L,), ss_all[lb * LB + i], jnp.float32)
             for i in range(LB)]

      for j in range(CPW):
        off = base + j * L
        v0 = vec_v[0, pl.ds(off, L)]
        v1 = vec_v[1, pl.ds(off, L)]
        v2 = vec_v[2, pl.ds(off, L)]
        cm = colmin_v[pl.ds(off, L)]
        for i in range(LB):
          t = sss[i] + a0s[i] * v0 + a1s[i] * v1 + a2s[i] * v2
          cm = jnp.minimum(cm, t)
        colmin_v[pl.ds(off, L)] = cm
    return 0

  lax.fori_loop(0, NCHUNK, outer, 0)


def _sc_body(tmpl_hbm, src_hbm, out_hbm, tmpl_v, src_v, colmin0_v, colmin1_v,
             out_v):
  core = lax.axis_index("c")
  sub = lax.axis_index("s")
  base = sub * (CPW * L)

  pltpu.sync_copy(tmpl_hbm.at[core], tmpl_v)
  pltpu.sync_copy(src_hbm.at[core], src_v)

  for j in range(CPW):
    off = base + j * L
    colmin0_v[pl.ds(off, L)] = jnp.full((L,), _INF, jnp.float32)
    colmin1_v[pl.ds(off, L)] = jnp.full((L,), _INF, jnp.float32)

  _direction_pass(src_v, tmpl_v, colmin0_v, base)
  _direction_pass(tmpl_v, src_v, colmin1_v, base)

  vacc = jnp.zeros((L,), jnp.float32)
  for j in range(CPW):
    off = base + j * L
    t0 = tmpl_v[0, pl.ds(off, L)]
    t1 = tmpl_v[1, pl.ds(off, L)]
    t2 = tmpl_v[2, pl.ds(off, L)]
    s0 = src_v[0, pl.ds(off, L)]
    s1 = src_v[1, pl.ds(off, L)]
    s2 = src_v[2, pl.ds(off, L)]
    min0 = colmin0_v[pl.ds(off, L)] + (t0 * t0 + t1 * t1 + t2 * t2)
    min1 = colmin1_v[pl.ds(off, L)] + (s0 * s0 + s1 * s1 + s2 * s2)
    vacc = vacc + (min0 + min1)

  out_v[...] = vacc
  pltpu.sync_copy(out_v, out_hbm.at[core, sub])


def _sc_chamfer(tmpl_t, src_t):
  mesh = plsc.VectorSubcoreMesh(core_axis_name="c", subcore_axis_name="s")
  run = pl.kernel(
      _sc_body,
      out_type=jax.ShapeDtypeStruct((B_SC, W_SUB, L), jnp.float32),
      mesh=mesh,
      scratch_types=[
          pltpu.VMEM((D, N), jnp.float32),
          pltpu.VMEM((D, N), jnp.float32),
          pltpu.VMEM((N,), jnp.float32),
          pltpu.VMEM((N,), jnp.float32),
          pltpu.VMEM((L,), jnp.float32),
      ],
  )
  out = run(tmpl_t, src_t)
  return jnp.sum(out, axis=(1, 2)) * (1.0 / N)


TB = 16


def _tc_body(x_ref, y_ref, out_ref):
  step = pl.program_id(0)
  acc = jnp.float32(0.0)
  for b in range(TB):
    lhs = x_ref[b]
    rhs = y_ref[b]

    r0 = lax.dot_general(lhs, rhs, (((0,), (0,)), ((), ())),
                         preferred_element_type=jnp.float32)

    m0 = jnp.min(r0, axis=1)
    m1 = jnp.min(r0, axis=0)
    acc = acc + (jnp.sum(m0) + jnp.sum(m1))

  scaled = acc * (1.0 / (N * B))

  @pl.when(step == 0)
  def _init():
    out_ref[0, 0, 0] = scaled

  @pl.when(step != 0)
  def _accum():
    out_ref[0, 0, 0] = out_ref[0, 0, 0] + scaled


def _tc_chamfer_mean(x, y):
  nb = x.shape[0]
  steps = nb // TB

  xt = jnp.transpose(x, (0, 2, 1))
  yt = jnp.transpose(y, (0, 2, 1))
  xx = jnp.sum(xt * xt, axis=1, keepdims=True)
  yy = jnp.sum(yt * yt, axis=1, keepdims=True)
  one = jnp.ones((nb, 1, N), jnp.float32)
  zero3 = jnp.zeros((nb, 3, N), jnp.float32)
  lhs = jnp.concatenate([xt * -2.0, one, xx, zero3], axis=1)
  rhs = jnp.concatenate([yt, yy, one, zero3], axis=1)

  out = pl.pallas_call(
      _tc_body,
      grid=(steps,),
      in_specs=[
          pl.BlockSpec((TB, 8, N), lambda i: (i, 0, 0)),
          pl.BlockSpec((TB, 8, N), lambda i: (i, 0, 0)),
      ],
      out_specs=pl.BlockSpec((1, 1, 1), lambda i: (0, 0, 0),
                             memory_space=pltpu.SMEM),
      out_shape=jax.ShapeDtypeStruct((1, 1, 1), jnp.float32),
      compiler_params=pltpu.CompilerParams(
          dimension_semantics=("arbitrary",),
      ),
  )(lhs, rhs)
  return jnp.reshape(out, ())


@jax.jit
def kernel(template, source):
  return _tc_chamfer_mean(template, source)
